# Initial kernel scaffold; baseline (speedup 1.0000x reference)
import jax
import jax.numpy as jnp
from jax.experimental import pallas as pl


def kernel(x, edge_index, edge_attr, query_embedding, W_in, b_in, W_q, b_q, Wl0, bl0, Wr0, g0, be0, Wl1, bl1, Wr1, g1, be1, W_att, b_att):
    raise NotImplementedError("write your pallas kernel here")


# speedup vs baseline: 2.9684x; 2.9684x over previous
import functools

import jax
import jax.numpy as jnp
from jax import lax
from jax.experimental import pallas as pl
from jax.experimental.pallas import tpu as pltpu
from jax.experimental.pallas import tpu_sc as plsc

N = 10000
DIN = 384
DH = 128
CL = 16
NC = 2
NS = 16
NW = NC * NS
K = 128
N_PAD = 10240
RPT = N_PAD // NS
BM = 1024


IB = 8


@functools.lru_cache(maxsize=None)
def _sc_agg_kernel(tot_ch):
    ch = tot_ch // NW
    mesh = plsc.VectorSubcoreMesh(core_axis_name="c", subcore_axis_name="s")

    @functools.partial(
        pl.kernel,
        out_type=jax.ShapeDtypeStruct((NC * N_PAD, DH), jnp.float32),
        mesh=mesh,
        scratch_types=[
            pltpu.VMEM_SHARED((N_PAD, DH), jnp.float32),
            pltpu.VMEM((IB, K), jnp.int32),
            pltpu.VMEM((IB, K), jnp.int32),
            pltpu.VMEM((K, DH), jnp.float32),
            pltpu.SemaphoreType.DMA,
        ],
    )
    def agg(h_hbm, src_hbm, dst_hbm, zrow_hbm,
            sums_hbm,
            acc, src_v, dst_v, rows_v, sem):
        c = lax.axis_index("c")
        s = lax.axis_index("s")
        wid = s * NC + c
        row0 = s * RPT

        pltpu.sync_copy(zrow_hbm, acc.at[pl.ds(row0, RPT)])
        plsc.subcore_barrier()

        base = wid * ch

        def outer(ob, carry):
            pltpu.sync_copy(src_hbm.at[pl.ds(base + ob * IB, IB)], src_v)
            pltpu.sync_copy(dst_hbm.at[pl.ds(base + ob * IB, IB)], dst_v)
            for j in range(IB):
                pltpu.async_copy(h_hbm.at[src_v.at[j]], rows_v, sem).wait()
                pltpu.sync_copy(rows_v, acc.at[dst_v.at[j]], add=True)
            return carry

        lax.fori_loop(0, ch // IB, outer, 0)
        plsc.subcore_barrier()

        out0 = c * N_PAD + row0
        pltpu.sync_copy(acc.at[pl.ds(row0, RPT)], sums_hbm.at[pl.ds(out0, RPT)])

    return agg


def _hist(dst_col):
    EB = 2048
    e_pad = dst_col.shape[0]

    def body(d_ref, o_ref):
        d = d_ref[...]
        hi = d // DH
        lo = d - hi * DH
        lane = lax.broadcasted_iota(jnp.int32, (1, DH), 1)
        oh_hi = (hi == lane).astype(jnp.bfloat16)
        oh_lo = (lo == lane).astype(jnp.bfloat16)
        part = lax.dot_general(oh_hi, oh_lo, (((0,), (0,)), ((), ())),
                               preferred_element_type=jnp.float32)

        @pl.when(pl.program_id(0) == 0)
        def _():
            o_ref[...] = jnp.zeros_like(o_ref)

        o_ref[...] += part

    return pl.pallas_call(
        body,
        grid=(e_pad // EB,),
        in_specs=[pl.BlockSpec((EB, 1), lambda i: (i, 0))],
        out_specs=pl.BlockSpec((DH, DH), lambda i: (0, 0)),
        out_shape=jax.ShapeDtypeStruct((DH, DH), jnp.float32),
    )(dst_col)


def _proj(x_pad, query, W_in, b_in, W_q, b_q):
    def body(x_ref, q_ref, wi_ref, bi_ref, wq_ref, bq_ref, o_ref):
        qrow = jnp.dot(q_ref[...], wq_ref[...],
                       preferred_element_type=jnp.float32) + bq_ref[...]
        o_ref[...] = (jnp.dot(x_ref[...], wi_ref[...],
                              preferred_element_type=jnp.float32)
                      + bi_ref[...] + qrow)

    return pl.pallas_call(
        body,
        grid=(N_PAD // BM,),
        in_specs=[
            pl.BlockSpec((BM, DIN), lambda i: (i, 0)),
            pl.BlockSpec((1, DIN), lambda i: (0, 0)),
            pl.BlockSpec((DIN, DH), lambda i: (0, 0)),
            pl.BlockSpec((1, DH), lambda i: (0, 0)),
            pl.BlockSpec((DIN, DH), lambda i: (0, 0)),
            pl.BlockSpec((1, DH), lambda i: (0, 0)),
        ],
        out_specs=pl.BlockSpec((BM, DH), lambda i: (i, 0)),
        out_shape=jax.ShapeDtypeStruct((N_PAD, DH), jnp.float32),
    )(x_pad, query.reshape(1, DIN), W_in, b_in.reshape(1, DH),
      W_q, b_q.reshape(1, DH))


def _layer(h, sums, cnts, Wl, bl, Wr, g, be, relu, Watt=None, batt=None):
    att = Watt is not None
    npb = N_PAD // BM

    def body(h_ref, s0_ref, s1_ref, c_ref,
             wl_ref, bl_ref, wr_ref, g_ref, be_ref, *rest):
        if att:
            watt_ref, batt_ref, o_ref, lg_ref = rest
        else:
            (o_ref,) = rest
        cnt = jnp.maximum(c_ref[...], 1.0)
        mean = (s0_ref[...] + s1_ref[...]) / cnt
        t = (jnp.dot(mean, wl_ref[...], preferred_element_type=jnp.float32)
             + bl_ref[...]
             + jnp.dot(h_ref[...], wr_ref[...],
                       preferred_element_type=jnp.float32))
        m = jnp.mean(t, axis=-1, keepdims=True)
        v = jnp.mean((t - m) * (t - m), axis=-1, keepdims=True)
        ln = (t - m) * lax.rsqrt(v + 1e-5) * g_ref[...] + be_ref[...]
        hn = h_ref[...] + ln
        if relu:
            hn = jnp.maximum(hn, 0.0)
        o_ref[...] = hn
        if att:
            lg = (jnp.dot(hn, watt_ref[...],
                          preferred_element_type=jnp.float32) + batt_ref[...])
            row = pl.program_id(0) * BM + lax.broadcasted_iota(
                jnp.int32, (BM, 1), 0)
            lg_ref[...] = jnp.where(row < N, lg, -1e30)

    in_specs = [
        pl.BlockSpec((BM, DH), lambda i: (i, 0)),
        pl.BlockSpec((BM, DH), lambda i: (i, 0)),
        pl.BlockSpec((BM, DH), lambda i: (i + npb, 0)),
        pl.BlockSpec((BM, 1), lambda i: (i, 0)),
        pl.BlockSpec((DH, DH), lambda i: (0, 0)),
        pl.BlockSpec((1, DH), lambda i: (0, 0)),
        pl.BlockSpec((DH, DH), lambda i: (0, 0)),
        pl.BlockSpec((1, DH), lambda i: (0, 0)),
        pl.BlockSpec((1, DH), lambda i: (0, 0)),
    ]
    args = [h, sums, sums, cnts, Wl, bl.reshape(1, DH), Wr,
            g.reshape(1, DH), be.reshape(1, DH)]
    out_specs = pl.BlockSpec((BM, DH), lambda i: (i, 0))
    out_shape = jax.ShapeDtypeStruct((N_PAD, DH), jnp.float32)
    if att:
        in_specs += [
            pl.BlockSpec((DH, 1), lambda i: (0, 0)),
            pl.BlockSpec((1, 1), lambda i: (0, 0)),
        ]
        args += [Watt, batt.reshape(1, 1)]
        out_specs = [out_specs, pl.BlockSpec((BM, 1), lambda i: (i, 0))]
        out_shape = [out_shape,
                     jax.ShapeDtypeStruct((N_PAD, 1), jnp.float32)]

    return pl.pallas_call(
        body,
        grid=(npb,),
        in_specs=in_specs,
        out_specs=out_specs,
        out_shape=out_shape,
    )(*args)


def _softmax(lg2d):
    def body(l_ref, o_ref):
        l = l_ref[...]
        m = jnp.max(l)
        e = jnp.exp(l - m)
        o_ref[...] = e / jnp.sum(e)

    return pl.pallas_call(
        body,
        out_shape=jax.ShapeDtypeStruct(lg2d.shape, jnp.float32),
    )(lg2d)


def kernel(x, edge_index, edge_attr, query_embedding,
           W_in, b_in, W_q, b_q,
           Wl0, bl0, Wr0, g0, be0,
           Wl1, bl1, Wr1, g1, be1,
           W_att, b_att):
    E = edge_index.shape[1]
    ch = -(-E // (NW * K))
    ch = -(-ch // 8) * 8
    tot_ch = ch * NW
    e_pad = tot_ch * K

    src = edge_index[0]
    dst = edge_index[1]
    pad = e_pad - E
    src_p = jnp.concatenate(
        [src, jnp.zeros((pad,), jnp.int32)]).reshape(tot_ch, K)
    dst_p = jnp.concatenate(
        [dst, jnp.full((pad,), N_PAD - 1, jnp.int32)]).reshape(tot_ch, K)
    x_pad = jnp.pad(x, ((0, N_PAD - N), (0, 0)))
    zrow = jnp.zeros((RPT, DH), jnp.float32)

    agg = _sc_agg_kernel(tot_ch)

    h0 = _proj(x_pad, query_embedding, W_in, b_in, W_q, b_q)
    cnt2d = _hist(dst_p.reshape(e_pad, 1))
    cnt_col = cnt2d.reshape(-1)[:N_PAD].reshape(N_PAD, 1)
    sums0 = agg(h0, src_p, dst_p, zrow)
    h1 = _layer(h0, sums0, cnt_col, Wl0, bl0, Wr0, g0, be0, relu=True)
    sums1 = agg(h1, src_p, dst_p, zrow)
    h2, lg = _layer(h1, sums1, cnt_col, Wl1, bl1, Wr1, g1, be1, relu=False,
                    Watt=W_att, batt=b_att)
    w = _softmax(lg.reshape(N_PAD // DH, DH))
    return h2[:N], w.reshape(-1)[:N]


# speedup vs baseline: 3.1992x; 1.0777x over previous
import functools

import jax
import jax.numpy as jnp
from jax import lax
from jax.experimental import pallas as pl
from jax.experimental.pallas import tpu as pltpu
from jax.experimental.pallas import tpu_sc as plsc

N = 10000
DIN = 384
DH = 128
CL = 16
NC = 2
NS = 16
NW = NC * NS
K = 128
N_PAD = 10240
RPT = N_PAD // NS
BM = 1024


IB = 8


@functools.lru_cache(maxsize=None)
def _sc_agg_kernel(tot_ch):
    ch = tot_ch // NW
    mesh = plsc.VectorSubcoreMesh(core_axis_name="c", subcore_axis_name="s")

    nb = ch // IB

    @functools.partial(
        pl.kernel,
        out_type=jax.ShapeDtypeStruct((NC * N_PAD, DH), jnp.float32),
        mesh=mesh,
        scratch_types=[
            pltpu.VMEM_SHARED((N_PAD, DH), jnp.float32),
            pltpu.VMEM((IB, K), jnp.int32),
            pltpu.VMEM((IB, K), jnp.int32),
            pltpu.VMEM((IB, K), jnp.int32),
            pltpu.VMEM((IB, K), jnp.int32),
            pltpu.VMEM((K, DH), jnp.float32),
            pltpu.VMEM((K, DH), jnp.float32),
            pltpu.SemaphoreType.DMA,
            pltpu.SemaphoreType.DMA,
            pltpu.SemaphoreType.DMA,
            pltpu.SemaphoreType.DMA,
        ],
    )
    def agg(h_hbm, src_hbm, dst_hbm, zrow_hbm,
            sums_hbm,
            acc, srcA, dstA, srcB, dstB, rows0, rows1,
            sem0, sem1, semA, semB):
        c = lax.axis_index("c")
        s = lax.axis_index("s")
        wid = s * NC + c
        row0 = s * RPT

        pltpu.sync_copy(zrow_hbm, acc.at[pl.ds(row0, RPT)])
        plsc.subcore_barrier()

        base = wid * ch
        rows = (rows0, rows1)
        sems = (sem0, sem1)

        pltpu.sync_copy(src_hbm.at[pl.ds(base, IB)], srcA)
        pltpu.sync_copy(dst_hbm.at[pl.ds(base, IB)], dstA)
        pltpu.async_copy(h_hbm.at[srcA.at[0]], rows0, sem0)

        def do_block(ob, Xs, Xd, Ys, Yd, semY):
            @pl.when(ob + 1 < nb)
            def _():
                off = base + (ob + 1) * IB
                pltpu.async_copy(src_hbm.at[pl.ds(off, IB)], Ys, semY)
                pltpu.async_copy(dst_hbm.at[pl.ds(off, IB)], Yd, semY)

            for t in range(IB):
                buf, sem = rows[t % 2], sems[t % 2]
                nbuf, nsem = rows[(t + 1) % 2], sems[(t + 1) % 2]
                pltpu.make_async_copy(h_hbm.at[Xs.at[t]], buf, sem).wait()
                if t + 1 < IB:
                    pltpu.async_copy(h_hbm.at[Xs.at[t + 1]], nbuf, nsem)
                else:
                    @pl.when(ob + 1 < nb)
                    def _():
                        off = base + (ob + 1) * IB
                        pltpu.make_async_copy(
                            src_hbm.at[pl.ds(off, IB)], Ys, semY).wait()
                        pltpu.make_async_copy(
                            dst_hbm.at[pl.ds(off, IB)], Yd, semY).wait()
                        pltpu.async_copy(h_hbm.at[Ys.at[0]], nbuf, nsem)
                pltpu.sync_copy(buf, acc.at[Xd.at[t]], add=True)

        def pair(p, carry):
            do_block(2 * p, srcA, dstA, srcB, dstB, semB)
            do_block(2 * p + 1, srcB, dstB, srcA, dstA, semA)
            return carry

        lax.fori_loop(0, nb // 2, pair, 0)
        plsc.subcore_barrier()

        out0 = c * N_PAD + row0
        pltpu.sync_copy(acc.at[pl.ds(row0, RPT)], sums_hbm.at[pl.ds(out0, RPT)])

    return agg


def _hist(dst_col):
    EB = 2048
    e_pad = dst_col.shape[0]

    def body(d_ref, o_ref):
        d = d_ref[...]
        hi = d // DH
        lo = d - hi * DH
        lane = lax.broadcasted_iota(jnp.int32, (1, DH), 1)
        oh_hi = (hi == lane).astype(jnp.bfloat16)
        oh_lo = (lo == lane).astype(jnp.bfloat16)
        part = lax.dot_general(oh_hi, oh_lo, (((0,), (0,)), ((), ())),
                               preferred_element_type=jnp.float32)

        @pl.when(pl.program_id(0) == 0)
        def _():
            o_ref[...] = jnp.zeros_like(o_ref)

        o_ref[...] += part

    return pl.pallas_call(
        body,
        grid=(e_pad // EB,),
        in_specs=[pl.BlockSpec((EB, 1), lambda i: (i, 0))],
        out_specs=pl.BlockSpec((DH, DH), lambda i: (0, 0)),
        out_shape=jax.ShapeDtypeStruct((DH, DH), jnp.float32),
    )(dst_col)


def _proj(x_pad, query, W_in, b_in, W_q, b_q):
    def body(x_ref, q_ref, wi_ref, bi_ref, wq_ref, bq_ref, o_ref):
        qrow = jnp.dot(q_ref[...], wq_ref[...],
                       preferred_element_type=jnp.float32) + bq_ref[...]
        o_ref[...] = (jnp.dot(x_ref[...], wi_ref[...],
                              preferred_element_type=jnp.float32)
                      + bi_ref[...] + qrow)

    return pl.pallas_call(
        body,
        grid=(N_PAD // BM,),
        in_specs=[
            pl.BlockSpec((BM, DIN), lambda i: (i, 0)),
            pl.BlockSpec((1, DIN), lambda i: (0, 0)),
            pl.BlockSpec((DIN, DH), lambda i: (0, 0)),
            pl.BlockSpec((1, DH), lambda i: (0, 0)),
            pl.BlockSpec((DIN, DH), lambda i: (0, 0)),
            pl.BlockSpec((1, DH), lambda i: (0, 0)),
        ],
        out_specs=pl.BlockSpec((BM, DH), lambda i: (i, 0)),
        out_shape=jax.ShapeDtypeStruct((N_PAD, DH), jnp.float32),
    )(x_pad, query.reshape(1, DIN), W_in, b_in.reshape(1, DH),
      W_q, b_q.reshape(1, DH))


def _layer(h, sums, cnts, Wl, bl, Wr, g, be, relu, Watt=None, batt=None):
    att = Watt is not None
    npb = N_PAD // BM

    def body(h_ref, s0_ref, s1_ref, c_ref,
             wl_ref, bl_ref, wr_ref, g_ref, be_ref, *rest):
        if att:
            watt_ref, batt_ref, o_ref, lg_ref = rest
        else:
            (o_ref,) = rest
        cnt = jnp.maximum(c_ref[...], 1.0)
        mean = (s0_ref[...] + s1_ref[...]) / cnt
        t = (jnp.dot(mean, wl_ref[...], preferred_element_type=jnp.float32)
             + bl_ref[...]
             + jnp.dot(h_ref[...], wr_ref[...],
                       preferred_element_type=jnp.float32))
        m = jnp.mean(t, axis=-1, keepdims=True)
        v = jnp.mean((t - m) * (t - m), axis=-1, keepdims=True)
        ln = (t - m) * lax.rsqrt(v + 1e-5) * g_ref[...] + be_ref[...]
        hn = h_ref[...] + ln
        if relu:
            hn = jnp.maximum(hn, 0.0)
        o_ref[...] = hn
        if att:
            lg = (jnp.dot(hn, watt_ref[...],
                          preferred_element_type=jnp.float32) + batt_ref[...])
            row = pl.program_id(0) * BM + lax.broadcasted_iota(
                jnp.int32, (BM, 1), 0)
            lg_ref[...] = jnp.where(row < N, lg, -1e30)

    in_specs = [
        pl.BlockSpec((BM, DH), lambda i: (i, 0)),
        pl.BlockSpec((BM, DH), lambda i: (i, 0)),
        pl.BlockSpec((BM, DH), lambda i: (i + npb, 0)),
        pl.BlockSpec((BM, 1), lambda i: (i, 0)),
        pl.BlockSpec((DH, DH), lambda i: (0, 0)),
        pl.BlockSpec((1, DH), lambda i: (0, 0)),
        pl.BlockSpec((DH, DH), lambda i: (0, 0)),
        pl.BlockSpec((1, DH), lambda i: (0, 0)),
        pl.BlockSpec((1, DH), lambda i: (0, 0)),
    ]
    args = [h, sums, sums, cnts, Wl, bl.reshape(1, DH), Wr,
            g.reshape(1, DH), be.reshape(1, DH)]
    out_specs = pl.BlockSpec((BM, DH), lambda i: (i, 0))
    out_shape = jax.ShapeDtypeStruct((N_PAD, DH), jnp.float32)
    if att:
        in_specs += [
            pl.BlockSpec((DH, 1), lambda i: (0, 0)),
            pl.BlockSpec((1, 1), lambda i: (0, 0)),
        ]
        args += [Watt, batt.reshape(1, 1)]
        out_specs = [out_specs, pl.BlockSpec((BM, 1), lambda i: (i, 0))]
        out_shape = [out_shape,
                     jax.ShapeDtypeStruct((N_PAD, 1), jnp.float32)]

    return pl.pallas_call(
        body,
        grid=(npb,),
        in_specs=in_specs,
        out_specs=out_specs,
        out_shape=out_shape,
    )(*args)


def _softmax(lg2d):
    def body(l_ref, o_ref):
        l = l_ref[...]
        m = jnp.max(l)
        e = jnp.exp(l - m)
        o_ref[...] = e / jnp.sum(e)

    return pl.pallas_call(
        body,
        out_shape=jax.ShapeDtypeStruct(lg2d.shape, jnp.float32),
    )(lg2d)


def kernel(x, edge_index, edge_attr, query_embedding,
           W_in, b_in, W_q, b_q,
           Wl0, bl0, Wr0, g0, be0,
           Wl1, bl1, Wr1, g1, be1,
           W_att, b_att):
    E = edge_index.shape[1]
    ch = -(-E // (NW * K))
    ch = -(-ch // 16) * 16
    tot_ch = ch * NW
    e_pad = tot_ch * K

    src = edge_index[0]
    dst = edge_index[1]
    pad = e_pad - E
    src_p = jnp.concatenate(
        [src, jnp.zeros((pad,), jnp.int32)]).reshape(tot_ch, K)
    dst_p = jnp.concatenate(
        [dst, jnp.full((pad,), N_PAD - 1, jnp.int32)]).reshape(tot_ch, K)
    x_pad = jnp.pad(x, ((0, N_PAD - N), (0, 0)))
    zrow = jnp.zeros((RPT, DH), jnp.float32)

    agg = _sc_agg_kernel(tot_ch)

    h0 = _proj(x_pad, query_embedding, W_in, b_in, W_q, b_q)
    cnt2d = _hist(dst_p.reshape(e_pad, 1))
    cnt_col = cnt2d.reshape(-1)[:N_PAD].reshape(N_PAD, 1)
    sums0 = agg(h0, src_p, dst_p, zrow)
    h1 = _layer(h0, sums0, cnt_col, Wl0, bl0, Wr0, g0, be0, relu=True)
    sums1 = agg(h1, src_p, dst_p, zrow)
    h2, lg = _layer(h1, sums1, cnt_col, Wl1, bl1, Wr1, g1, be1, relu=False,
                    Watt=W_att, batt=b_att)
    w = _softmax(lg.reshape(N_PAD // DH, DH))
    return h2[:N], w.reshape(-1)[:N]


# speedup vs baseline: 6.6403x; 2.0756x over previous
import functools

import jax
import jax.numpy as jnp
from jax import lax
from jax.experimental import pallas as pl
from jax.experimental.pallas import tpu as pltpu
from jax.experimental.pallas import tpu_sc as plsc

N = 10000
DIN = 384
DH = 128
CL = 16
NC = 2
NS = 16
NW = NC * NS
K = 128
N_PAD = 10240
RPT = N_PAD // NS
BM = 1024


IB = 8


@functools.lru_cache(maxsize=None)
def _sc_agg_kernel(tot_ch):
    ch = tot_ch // NW
    mesh = plsc.VectorSubcoreMesh(core_axis_name="c", subcore_axis_name="s")

    nb = ch // IB

    @functools.partial(
        pl.kernel,
        out_type=jax.ShapeDtypeStruct((NC * N_PAD, DH), jnp.float32),
        mesh=mesh,
        scratch_types=[
            pltpu.VMEM_SHARED((N_PAD, DH), jnp.float32),
            pltpu.VMEM((IB, K), jnp.int32),
            pltpu.VMEM((IB, K), jnp.int32),
            pltpu.VMEM((IB, K), jnp.int32),
            pltpu.VMEM((IB, K), jnp.int32),
            pltpu.VMEM((K, DH), jnp.float32),
            pltpu.VMEM((K, DH), jnp.float32),
            pltpu.SemaphoreType.DMA,
            pltpu.SemaphoreType.DMA,
            pltpu.SemaphoreType.DMA,
            pltpu.SemaphoreType.DMA,
        ],
    )
    def agg(h_hbm, src_hbm, dst_hbm, zrow_hbm,
            sums_hbm,
            acc, srcA, dstA, srcB, dstB, rows0, rows1,
            sem0, sem1, semA, semB):
        c = lax.axis_index("c")
        s = lax.axis_index("s")
        wid = s * NC + c
        row0 = s * RPT

        pltpu.sync_copy(zrow_hbm, acc.at[pl.ds(row0, RPT)])
        plsc.subcore_barrier()

        base = wid * ch
        rows = (rows0, rows1)
        sems = (sem0, sem1)

        pltpu.sync_copy(src_hbm.at[pl.ds(base, IB)], srcA)
        pltpu.sync_copy(dst_hbm.at[pl.ds(base, IB)], dstA)
        pltpu.async_copy(h_hbm.at[srcA.at[0]], rows0, sem0)

        def do_block(ob, Xs, Xd, Ys, Yd, semY):
            @pl.when(ob + 1 < nb)
            def _():
                off = base + (ob + 1) * IB
                pltpu.async_copy(src_hbm.at[pl.ds(off, IB)], Ys, semY)
                pltpu.async_copy(dst_hbm.at[pl.ds(off, IB)], Yd, semY)

            for t in range(IB):
                buf, sem = rows[t % 2], sems[t % 2]
                nbuf, nsem = rows[(t + 1) % 2], sems[(t + 1) % 2]
                pltpu.make_async_copy(h_hbm.at[Xs.at[t]], buf, sem).wait()
                if t + 1 < IB:
                    pltpu.async_copy(h_hbm.at[Xs.at[t + 1]], nbuf, nsem)
                else:
                    @pl.when(ob + 1 < nb)
                    def _():
                        off = base + (ob + 1) * IB
                        pltpu.make_async_copy(
                            src_hbm.at[pl.ds(off, IB)], Ys, semY).wait()
                        pltpu.make_async_copy(
                            dst_hbm.at[pl.ds(off, IB)], Yd, semY).wait()
                        pltpu.async_copy(h_hbm.at[Ys.at[0]], nbuf, nsem)
                pltpu.sync_copy(buf, acc.at[Xd.at[t]], add=True)

        def pair(p, carry):
            do_block(2 * p, srcA, dstA, srcB, dstB, semB)
            do_block(2 * p + 1, srcB, dstB, srcA, dstA, semA)
            return carry

        lax.fori_loop(0, nb // 2, pair, 0)
        plsc.subcore_barrier()

        out0 = c * N_PAD + row0
        pltpu.sync_copy(acc.at[pl.ds(row0, RPT)], sums_hbm.at[pl.ds(out0, RPT)])

    return agg


def _hist(dst_col):
    EB = 2048
    e_pad = dst_col.shape[0]

    def body(d_ref, o_ref):
        d = d_ref[...]
        hi = d // DH
        lo = d - hi * DH
        lane = lax.broadcasted_iota(jnp.int32, (1, DH), 1)
        oh_hi = (hi == lane).astype(jnp.bfloat16)
        oh_lo = (lo == lane).astype(jnp.bfloat16)
        part = lax.dot_general(oh_hi, oh_lo, (((0,), (0,)), ((), ())),
                               preferred_element_type=jnp.float32)

        @pl.when(pl.program_id(0) == 0)
        def _():
            o_ref[...] = jnp.zeros_like(o_ref)

        o_ref[...] += part

    return pl.pallas_call(
        body,
        grid=(e_pad // EB,),
        in_specs=[pl.BlockSpec((EB, 1), lambda i: (i, 0))],
        out_specs=pl.BlockSpec((DH, DH), lambda i: (0, 0)),
        out_shape=jax.ShapeDtypeStruct((DH, DH), jnp.float32),
    )(dst_col)


def _proj(x_pad, query, W_in, b_in, W_q, b_q):
    def body(x_ref, q_ref, wi_ref, bi_ref, wq_ref, bq_ref, o_ref):
        qrow = jnp.dot(q_ref[...], wq_ref[...],
                       preferred_element_type=jnp.float32) + bq_ref[...]
        o_ref[...] = (jnp.dot(x_ref[...], wi_ref[...],
                              preferred_element_type=jnp.float32)
                      + bi_ref[...] + qrow)

    return pl.pallas_call(
        body,
        grid=(N_PAD // BM,),
        in_specs=[
            pl.BlockSpec((BM, DIN), lambda i: (i, 0)),
            pl.BlockSpec((1, DIN), lambda i: (0, 0)),
            pl.BlockSpec((DIN, DH), lambda i: (0, 0)),
            pl.BlockSpec((1, DH), lambda i: (0, 0)),
            pl.BlockSpec((DIN, DH), lambda i: (0, 0)),
            pl.BlockSpec((1, DH), lambda i: (0, 0)),
        ],
        out_specs=pl.BlockSpec((BM, DH), lambda i: (i, 0)),
        out_shape=jax.ShapeDtypeStruct((N_PAD, DH), jnp.float32),
    )(x_pad, query.reshape(1, DIN), W_in, b_in.reshape(1, DH),
      W_q, b_q.reshape(1, DH))


def _layer(h, sums, cnts, Wl, bl, Wr, g, be, relu, Watt=None, batt=None):
    att = Watt is not None
    npb = N_PAD // BM

    def body(h_ref, s0_ref, s1_ref, c_ref,
             wl_ref, bl_ref, wr_ref, g_ref, be_ref, *rest):
        if att:
            watt_ref, batt_ref, o_ref, lg_ref = rest
        else:
            (o_ref,) = rest
        cnt = jnp.maximum(c_ref[...], 1.0)
        mean = (s0_ref[...] + s1_ref[...]) / cnt
        t = (jnp.dot(mean, wl_ref[...], preferred_element_type=jnp.float32)
             + bl_ref[...]
             + jnp.dot(h_ref[...], wr_ref[...],
                       preferred_element_type=jnp.float32))
        m = jnp.mean(t, axis=-1, keepdims=True)
        v = jnp.mean((t - m) * (t - m), axis=-1, keepdims=True)
        ln = (t - m) * lax.rsqrt(v + 1e-5) * g_ref[...] + be_ref[...]
        hn = h_ref[...] + ln
        if relu:
            hn = jnp.maximum(hn, 0.0)
        o_ref[...] = hn
        if att:
            lg = (jnp.dot(hn, watt_ref[...],
                          preferred_element_type=jnp.float32) + batt_ref[...])
            row = pl.program_id(0) * BM + lax.broadcasted_iota(
                jnp.int32, (BM, 1), 0)
            lg_ref[...] = jnp.where(row < N, lg, -1e30)

    in_specs = [
        pl.BlockSpec((BM, DH), lambda i: (i, 0)),
        pl.BlockSpec((BM, DH), lambda i: (i, 0)),
        pl.BlockSpec((BM, DH), lambda i: (i + npb, 0)),
        pl.BlockSpec((BM, 1), lambda i: (i, 0)),
        pl.BlockSpec((DH, DH), lambda i: (0, 0)),
        pl.BlockSpec((1, DH), lambda i: (0, 0)),
        pl.BlockSpec((DH, DH), lambda i: (0, 0)),
        pl.BlockSpec((1, DH), lambda i: (0, 0)),
        pl.BlockSpec((1, DH), lambda i: (0, 0)),
    ]
    args = [h, sums, sums, cnts, Wl, bl.reshape(1, DH), Wr,
            g.reshape(1, DH), be.reshape(1, DH)]
    out_specs = pl.BlockSpec((BM, DH), lambda i: (i, 0))
    out_shape = jax.ShapeDtypeStruct((N_PAD, DH), jnp.float32)
    if att:
        in_specs += [
            pl.BlockSpec((DH, 1), lambda i: (0, 0)),
            pl.BlockSpec((1, 1), lambda i: (0, 0)),
        ]
        args += [Watt, batt.reshape(1, 1)]
        out_specs = [out_specs, pl.BlockSpec((BM, 1), lambda i: (i, 0))]
        out_shape = [out_shape,
                     jax.ShapeDtypeStruct((N_PAD, 1), jnp.float32)]

    return pl.pallas_call(
        body,
        grid=(npb,),
        in_specs=in_specs,
        out_specs=out_specs,
        out_shape=out_shape,
    )(*args)


def _softmax(lg2d):
    def body(l_ref, o_ref):
        l = l_ref[...]
        m = jnp.max(l)
        e = jnp.exp(l - m)
        o_ref[...] = e / jnp.sum(e)

    return pl.pallas_call(
        body,
        out_shape=jax.ShapeDtypeStruct(lg2d.shape, jnp.float32),
    )(lg2d)


def kernel(x, edge_index, edge_attr, query_embedding,
           W_in, b_in, W_q, b_q,
           Wl0, bl0, Wr0, g0, be0,
           Wl1, bl1, Wr1, g1, be1,
           W_att, b_att):
    E = edge_index.shape[1]
    ch = -(-E // (NW * K))
    ch = -(-ch // 16) * 16
    tot_ch = ch * NW
    e_pad = tot_ch * K

    src = edge_index[0]
    dst = edge_index[1]
    pad = e_pad - E
    pad_ids = lax.iota(jnp.int32, pad)
    src_p = jnp.concatenate(
        [src, pad_ids % N]).reshape(tot_ch, K)
    dst_p = jnp.concatenate(
        [dst, N + pad_ids % (N_PAD - N)]).reshape(tot_ch, K)
    x_pad = jnp.pad(x, ((0, N_PAD - N), (0, 0)))
    zrow = jnp.zeros((RPT, DH), jnp.float32)

    agg = _sc_agg_kernel(tot_ch)

    h0 = _proj(x_pad, query_embedding, W_in, b_in, W_q, b_q)
    cnt2d = _hist(dst_p.reshape(e_pad, 1))
    cnt_col = cnt2d.reshape(-1)[:N_PAD].reshape(N_PAD, 1)
    sums0 = agg(h0, src_p, dst_p, zrow)
    h1 = _layer(h0, sums0, cnt_col, Wl0, bl0, Wr0, g0, be0, relu=True)
    sums1 = agg(h1, src_p, dst_p, zrow)
    h2, lg = _layer(h1, sums1, cnt_col, Wl1, bl1, Wr1, g1, be1, relu=False,
                    Watt=W_att, batt=b_att)
    w = _softmax(lg.reshape(N_PAD // DH, DH))
    return h2[:N], w.reshape(-1)[:N]


# speedup vs baseline: 7.0528x; 1.0621x over previous
import functools

import jax
import jax.numpy as jnp
from jax import lax
from jax.experimental import pallas as pl
from jax.experimental.pallas import tpu as pltpu
from jax.experimental.pallas import tpu_sc as plsc

N = 10000
DIN = 384
DH = 128
CL = 16
NC = 2
NS = 16
NW = NC * NS
K = 128
N_PAD = 10240
RPT = N_PAD // NS
BM = 1024


IB = 8


@functools.lru_cache(maxsize=None)
def _sc_agg_kernel(tot_ch):
    ch = tot_ch // NW
    mesh = plsc.VectorSubcoreMesh(core_axis_name="c", subcore_axis_name="s")

    nb = ch // IB

    @functools.partial(
        pl.kernel,
        out_type=jax.ShapeDtypeStruct((NC * N_PAD, DH), jnp.float32),
        mesh=mesh,
        scratch_types=[
            pltpu.VMEM_SHARED((N_PAD, DH), jnp.float32),
            pltpu.VMEM((IB, K), jnp.int32),
            pltpu.VMEM((IB, K), jnp.int32),
            pltpu.VMEM((IB, K), jnp.int32),
            pltpu.VMEM((IB, K), jnp.int32),
            pltpu.VMEM((K, DH), jnp.float32),
            pltpu.VMEM((K, DH), jnp.float32),
            pltpu.SemaphoreType.DMA,
            pltpu.SemaphoreType.DMA,
            pltpu.SemaphoreType.DMA,
            pltpu.SemaphoreType.DMA,
        ],
    )
    def agg(h_hbm, src_hbm, dst_hbm, zrow_hbm,
            sums_hbm,
            acc, srcA, dstA, srcB, dstB, rows0, rows1,
            sem0, sem1, semA, semB):
        c = lax.axis_index("c")
        s = lax.axis_index("s")
        wid = s * NC + c
        row0 = s * RPT

        pltpu.sync_copy(zrow_hbm, acc.at[pl.ds(row0, RPT)])
        plsc.subcore_barrier()

        base = wid * ch
        rows = (rows0, rows1)
        sems = (sem0, sem1)

        pltpu.sync_copy(src_hbm.at[pl.ds(base, IB)], srcA)
        pltpu.sync_copy(dst_hbm.at[pl.ds(base, IB)], dstA)
        pltpu.async_copy(h_hbm.at[srcA.at[0]], rows0, sem0)

        def do_block(ob, Xs, Xd, Ys, Yd, semY):
            @pl.when(ob + 1 < nb)
            def _():
                off = base + (ob + 1) * IB
                pltpu.async_copy(src_hbm.at[pl.ds(off, IB)], Ys, semY)
                pltpu.async_copy(dst_hbm.at[pl.ds(off, IB)], Yd, semY)

            for t in range(IB):
                buf, sem = rows[t % 2], sems[t % 2]
                nbuf, nsem = rows[(t + 1) % 2], sems[(t + 1) % 2]
                pltpu.make_async_copy(h_hbm.at[Xs.at[t]], buf, sem).wait()
                if t + 1 < IB:
                    pltpu.async_copy(h_hbm.at[Xs.at[t + 1]], nbuf, nsem)
                else:
                    @pl.when(ob + 1 < nb)
                    def _():
                        off = base + (ob + 1) * IB
                        pltpu.make_async_copy(
                            src_hbm.at[pl.ds(off, IB)], Ys, semY).wait()
                        pltpu.make_async_copy(
                            dst_hbm.at[pl.ds(off, IB)], Yd, semY).wait()
                        pltpu.async_copy(h_hbm.at[Ys.at[0]], nbuf, nsem)
                pltpu.sync_copy(buf, acc.at[Xd.at[t]], add=True)

        def pair(p, carry):
            do_block(2 * p, srcA, dstA, srcB, dstB, semB)
            do_block(2 * p + 1, srcB, dstB, srcA, dstA, semA)
            return carry

        lax.fori_loop(0, nb // 2, pair, 0)
        plsc.subcore_barrier()

        out0 = c * N_PAD + row0
        pltpu.sync_copy(acc.at[pl.ds(row0, RPT)], sums_hbm.at[pl.ds(out0, RPT)])

    return agg


def _hist(dst_col):
    EB = 8192
    e_pad = dst_col.shape[0]

    def body(d_ref, o_ref):
        d = d_ref[...]
        hi = d // DH
        lo = d - hi * DH
        lane = lax.broadcasted_iota(jnp.int32, (1, DH), 1)
        oh_hi = (hi == lane).astype(jnp.bfloat16)
        oh_lo = (lo == lane).astype(jnp.bfloat16)
        part = lax.dot_general(oh_hi, oh_lo, (((0,), (0,)), ((), ())),
                               preferred_element_type=jnp.float32)

        @pl.when(pl.program_id(0) == 0)
        def _():
            o_ref[...] = jnp.zeros_like(o_ref)

        o_ref[...] += part

    return pl.pallas_call(
        body,
        grid=(e_pad // EB,),
        in_specs=[pl.BlockSpec((EB, 1), lambda i: (i, 0))],
        out_specs=pl.BlockSpec((DH, DH), lambda i: (0, 0)),
        out_shape=jax.ShapeDtypeStruct((DH, DH), jnp.float32),
    )(dst_col)


def _proj(x_pad, query, W_in, b_in, W_q, b_q):
    def body(x_ref, q_ref, wi_ref, bi_ref, wq_ref, bq_ref, o_ref):
        qrow = jnp.dot(q_ref[...], wq_ref[...],
                       preferred_element_type=jnp.float32) + bq_ref[...]
        o_ref[...] = (jnp.dot(x_ref[...], wi_ref[...],
                              preferred_element_type=jnp.float32)
                      + bi_ref[...] + qrow)

    return pl.pallas_call(
        body,
        grid=(N_PAD // BM,),
        in_specs=[
            pl.BlockSpec((BM, DIN), lambda i: (i, 0)),
            pl.BlockSpec((1, DIN), lambda i: (0, 0)),
            pl.BlockSpec((DIN, DH), lambda i: (0, 0)),
            pl.BlockSpec((1, DH), lambda i: (0, 0)),
            pl.BlockSpec((DIN, DH), lambda i: (0, 0)),
            pl.BlockSpec((1, DH), lambda i: (0, 0)),
        ],
        out_specs=pl.BlockSpec((BM, DH), lambda i: (i, 0)),
        out_shape=jax.ShapeDtypeStruct((N_PAD, DH), jnp.float32),
    )(x_pad, query.reshape(1, DIN), W_in, b_in.reshape(1, DH),
      W_q, b_q.reshape(1, DH))


def _layer(h, sums, cnts, Wl, bl, Wr, g, be, relu, Watt=None, batt=None):
    att = Watt is not None
    npb = N_PAD // BM

    def body(h_ref, s0_ref, s1_ref, c_ref,
             wl_ref, bl_ref, wr_ref, g_ref, be_ref, *rest):
        if att:
            watt_ref, batt_ref, o_ref, lg_ref = rest
        else:
            (o_ref,) = rest
        cnt = jnp.maximum(c_ref[...], 1.0)
        mean = (s0_ref[...] + s1_ref[...]) / cnt
        t = (jnp.dot(mean, wl_ref[...], preferred_element_type=jnp.float32)
             + bl_ref[...]
             + jnp.dot(h_ref[...], wr_ref[...],
                       preferred_element_type=jnp.float32))
        m = jnp.mean(t, axis=-1, keepdims=True)
        v = jnp.mean((t - m) * (t - m), axis=-1, keepdims=True)
        ln = (t - m) * lax.rsqrt(v + 1e-5) * g_ref[...] + be_ref[...]
        hn = h_ref[...] + ln
        if relu:
            hn = jnp.maximum(hn, 0.0)
        o_ref[...] = hn
        if att:
            lg = (jnp.dot(hn, watt_ref[...],
                          preferred_element_type=jnp.float32) + batt_ref[...])
            row = pl.program_id(0) * BM + lax.broadcasted_iota(
                jnp.int32, (BM, 1), 0)
            lg_ref[...] = jnp.where(row < N, lg, -1e30)

    in_specs = [
        pl.BlockSpec((BM, DH), lambda i: (i, 0)),
        pl.BlockSpec((BM, DH), lambda i: (i, 0)),
        pl.BlockSpec((BM, DH), lambda i: (i + npb, 0)),
        pl.BlockSpec((BM, 1), lambda i: (i, 0)),
        pl.BlockSpec((DH, DH), lambda i: (0, 0)),
        pl.BlockSpec((1, DH), lambda i: (0, 0)),
        pl.BlockSpec((DH, DH), lambda i: (0, 0)),
        pl.BlockSpec((1, DH), lambda i: (0, 0)),
        pl.BlockSpec((1, DH), lambda i: (0, 0)),
    ]
    args = [h, sums, sums, cnts, Wl, bl.reshape(1, DH), Wr,
            g.reshape(1, DH), be.reshape(1, DH)]
    out_specs = pl.BlockSpec((BM, DH), lambda i: (i, 0))
    out_shape = jax.ShapeDtypeStruct((N_PAD, DH), jnp.float32)
    if att:
        in_specs += [
            pl.BlockSpec((DH, 1), lambda i: (0, 0)),
            pl.BlockSpec((1, 1), lambda i: (0, 0)),
        ]
        args += [Watt, batt.reshape(1, 1)]
        out_specs = [out_specs, pl.BlockSpec((BM, 1), lambda i: (i, 0))]
        out_shape = [out_shape,
                     jax.ShapeDtypeStruct((N_PAD, 1), jnp.float32)]

    return pl.pallas_call(
        body,
        grid=(npb,),
        in_specs=in_specs,
        out_specs=out_specs,
        out_shape=out_shape,
    )(*args)


def _softmax(lg2d):
    def body(l_ref, o_ref):
        l = l_ref[...]
        m = jnp.max(l)
        e = jnp.exp(l - m)
        o_ref[...] = e / jnp.sum(e)

    return pl.pallas_call(
        body,
        out_shape=jax.ShapeDtypeStruct(lg2d.shape, jnp.float32),
    )(lg2d)


def kernel(x, edge_index, edge_attr, query_embedding,
           W_in, b_in, W_q, b_q,
           Wl0, bl0, Wr0, g0, be0,
           Wl1, bl1, Wr1, g1, be1,
           W_att, b_att):
    E = edge_index.shape[1]
    ch = -(-E // (NW * K))
    ch = -(-ch // 16) * 16
    tot_ch = ch * NW
    e_pad = tot_ch * K

    src = edge_index[0]
    dst = edge_index[1]
    pad = e_pad - E
    pad_ids = lax.iota(jnp.int32, pad)
    src_p = jnp.concatenate(
        [src, pad_ids % N]).reshape(tot_ch, K)
    dst_p = jnp.concatenate(
        [dst, N + pad_ids % (N_PAD - N)]).reshape(tot_ch, K)
    x_pad = jnp.pad(x, ((0, N_PAD - N), (0, 0)))
    zrow = jnp.zeros((RPT, DH), jnp.float32)

    agg = _sc_agg_kernel(tot_ch)

    h0 = _proj(x_pad, query_embedding, W_in, b_in, W_q, b_q)
    sums0 = agg(h0, src_p, dst_p, zrow)
    cnt2d = _hist(dst_p.reshape(e_pad, 1))
    cnt_col = cnt2d.reshape(-1)[:N_PAD].reshape(N_PAD, 1)
    h1 = _layer(h0, sums0, cnt_col, Wl0, bl0, Wr0, g0, be0, relu=True)
    sums1 = agg(h1, src_p, dst_p, zrow)
    h2, lg = _layer(h1, sums1, cnt_col, Wl1, bl1, Wr1, g1, be1, relu=False,
                    Watt=W_att, batt=b_att)
    w = _softmax(lg.reshape(N_PAD // DH, DH))
    return h2[:N], w.reshape(-1)[:N]


# speedup vs baseline: 7.0589x; 1.0009x over previous
import functools

import jax
import jax.numpy as jnp
from jax import lax
from jax.experimental import pallas as pl
from jax.experimental.pallas import tpu as pltpu
from jax.experimental.pallas import tpu_sc as plsc

N = 10000
DIN = 384
DH = 128
NC = 2
NS = 16
NW = NC * NS
K = 128
N_PAD = 10240
RPT = N_PAD // NS
BM = 1024


IB = 8


@functools.lru_cache(maxsize=None)
def _sc_agg_kernel(tot_ch):
    ch = tot_ch // NW
    mesh = plsc.VectorSubcoreMesh(core_axis_name="c", subcore_axis_name="s")

    nb = ch // IB

    @functools.partial(
        pl.kernel,
        out_type=jax.ShapeDtypeStruct((NC * N_PAD, DH), jnp.float32),
        mesh=mesh,
        scratch_types=[
            pltpu.VMEM_SHARED((N_PAD, DH), jnp.float32),
            pltpu.VMEM((IB, K), jnp.int32),
            pltpu.VMEM((IB, K), jnp.int32),
            pltpu.VMEM((IB, K), jnp.int32),
            pltpu.VMEM((IB, K), jnp.int32),
            pltpu.VMEM((K, DH), jnp.float32),
            pltpu.VMEM((K, DH), jnp.float32),
            pltpu.SemaphoreType.DMA,
            pltpu.SemaphoreType.DMA,
            pltpu.SemaphoreType.DMA,
            pltpu.SemaphoreType.DMA,
        ],
    )
    def agg(h_hbm, src_hbm, dst_hbm, zrow_hbm,
            sums_hbm,
            acc, srcA, dstA, srcB, dstB, rows0, rows1,
            sem0, sem1, semA, semB):
        c = lax.axis_index("c")
        s = lax.axis_index("s")
        wid = s * NC + c
        row0 = s * RPT

        pltpu.sync_copy(zrow_hbm, acc.at[pl.ds(row0, RPT)])
        plsc.subcore_barrier()

        base = wid * ch
        rows = (rows0, rows1)
        sems = (sem0, sem1)

        pltpu.sync_copy(src_hbm.at[pl.ds(base, IB)], srcA)
        pltpu.sync_copy(dst_hbm.at[pl.ds(base, IB)], dstA)
        pltpu.async_copy(h_hbm.at[srcA.at[0]], rows0, sem0)

        def do_block(ob, Xs, Xd, Ys, Yd, semY):
            @pl.when(ob + 1 < nb)
            def _():
                off = base + (ob + 1) * IB
                pltpu.async_copy(src_hbm.at[pl.ds(off, IB)], Ys, semY)
                pltpu.async_copy(dst_hbm.at[pl.ds(off, IB)], Yd, semY)

            for t in range(IB):
                buf, sem = rows[t % 2], sems[t % 2]
                nbuf, nsem = rows[(t + 1) % 2], sems[(t + 1) % 2]
                pltpu.make_async_copy(h_hbm.at[Xs.at[t]], buf, sem).wait()
                if t + 1 < IB:
                    pltpu.async_copy(h_hbm.at[Xs.at[t + 1]], nbuf, nsem)
                else:
                    @pl.when(ob + 1 < nb)
                    def _():
                        off = base + (ob + 1) * IB
                        pltpu.make_async_copy(
                            src_hbm.at[pl.ds(off, IB)], Ys, semY).wait()
                        pltpu.make_async_copy(
                            dst_hbm.at[pl.ds(off, IB)], Yd, semY).wait()
                        pltpu.async_copy(h_hbm.at[Ys.at[0]], nbuf, nsem)
                pltpu.sync_copy(buf, acc.at[Xd.at[t]], add=True)

        def pair(p, carry):
            do_block(2 * p, srcA, dstA, srcB, dstB, semB)
            do_block(2 * p + 1, srcB, dstB, srcA, dstA, semA)
            return carry

        lax.fori_loop(0, nb // 2, pair, 0)
        plsc.subcore_barrier()

        out0 = c * N_PAD + row0
        pltpu.sync_copy(acc.at[pl.ds(row0, RPT)], sums_hbm.at[pl.ds(out0, RPT)])

    return agg


def _hist(dst_col):
    EB = 8192
    e_pad = dst_col.shape[0]

    def body(d_ref, o_ref):
        d = d_ref[...]
        hi = d // DH
        lo = d - hi * DH
        lane = lax.broadcasted_iota(jnp.int32, (1, DH), 1)
        oh_hi = (hi == lane).astype(jnp.bfloat16)
        oh_lo = (lo == lane).astype(jnp.bfloat16)
        part = lax.dot_general(oh_hi, oh_lo, (((0,), (0,)), ((), ())),
                               preferred_element_type=jnp.float32)

        @pl.when(pl.program_id(0) == 0)
        def _():
            o_ref[...] = jnp.zeros_like(o_ref)

        o_ref[...] += part

    return pl.pallas_call(
        body,
        grid=(e_pad // EB,),
        in_specs=[pl.BlockSpec((EB, 1), lambda i: (i, 0))],
        out_specs=pl.BlockSpec((DH, DH), lambda i: (0, 0)),
        out_shape=jax.ShapeDtypeStruct((DH, DH), jnp.float32),
    )(dst_col)


def _proj(x_pad, query, W_in, b_in, W_q, b_q):
    def body(x_ref, q_ref, wi_ref, bi_ref, wq_ref, bq_ref, o_ref):
        qrow = jnp.dot(q_ref[...], wq_ref[...],
                       preferred_element_type=jnp.float32) + bq_ref[...]
        o_ref[...] = (jnp.dot(x_ref[...], wi_ref[...],
                              preferred_element_type=jnp.float32)
                      + bi_ref[...] + qrow)

    return pl.pallas_call(
        body,
        grid=(N_PAD // BM,),
        in_specs=[
            pl.BlockSpec((BM, DIN), lambda i: (i, 0)),
            pl.BlockSpec((1, DIN), lambda i: (0, 0)),
            pl.BlockSpec((DIN, DH), lambda i: (0, 0)),
            pl.BlockSpec((1, DH), lambda i: (0, 0)),
            pl.BlockSpec((DIN, DH), lambda i: (0, 0)),
            pl.BlockSpec((1, DH), lambda i: (0, 0)),
        ],
        out_specs=pl.BlockSpec((BM, DH), lambda i: (i, 0)),
        out_shape=jax.ShapeDtypeStruct((N_PAD, DH), jnp.float32),
    )(x_pad, query.reshape(1, DIN), W_in, b_in.reshape(1, DH),
      W_q, b_q.reshape(1, DH))


def _layer(h, sums, cnts, Wl, bl, Wr, g, be, relu, Watt=None, batt=None):
    att = Watt is not None
    npb = N_PAD // BM

    def body(h_ref, s0_ref, s1_ref, c_ref,
             wl_ref, bl_ref, wr_ref, g_ref, be_ref, *rest):
        if att:
            watt_ref, batt_ref, o_ref, lg_ref = rest
        else:
            (o_ref,) = rest
        cnt = jnp.maximum(c_ref[...], 1.0)
        mean = (s0_ref[...] + s1_ref[...]) / cnt
        t = (jnp.dot(mean, wl_ref[...], preferred_element_type=jnp.float32)
             + bl_ref[...]
             + jnp.dot(h_ref[...], wr_ref[...],
                       preferred_element_type=jnp.float32))
        m = jnp.mean(t, axis=-1, keepdims=True)
        v = jnp.mean((t - m) * (t - m), axis=-1, keepdims=True)
        ln = (t - m) * lax.rsqrt(v + 1e-5) * g_ref[...] + be_ref[...]
        hn = h_ref[...] + ln
        if relu:
            hn = jnp.maximum(hn, 0.0)
        o_ref[...] = hn
        if att:
            lg = (jnp.dot(hn, watt_ref[...],
                          preferred_element_type=jnp.float32) + batt_ref[...])
            row = pl.program_id(0) * BM + lax.broadcasted_iota(
                jnp.int32, (BM, 1), 0)
            lg_ref[...] = jnp.where(row < N, lg, -1e30)

    in_specs = [
        pl.BlockSpec((BM, DH), lambda i: (i, 0)),
        pl.BlockSpec((BM, DH), lambda i: (i, 0)),
        pl.BlockSpec((BM, DH), lambda i: (i + npb, 0)),
        pl.BlockSpec((BM, 1), lambda i: (i, 0)),
        pl.BlockSpec((DH, DH), lambda i: (0, 0)),
        pl.BlockSpec((1, DH), lambda i: (0, 0)),
        pl.BlockSpec((DH, DH), lambda i: (0, 0)),
        pl.BlockSpec((1, DH), lambda i: (0, 0)),
        pl.BlockSpec((1, DH), lambda i: (0, 0)),
    ]
    args = [h, sums, sums, cnts, Wl, bl.reshape(1, DH), Wr,
            g.reshape(1, DH), be.reshape(1, DH)]
    out_specs = pl.BlockSpec((BM, DH), lambda i: (i, 0))
    out_shape = jax.ShapeDtypeStruct((N_PAD, DH), jnp.float32)
    if att:
        in_specs += [
            pl.BlockSpec((DH, 1), lambda i: (0, 0)),
            pl.BlockSpec((1, 1), lambda i: (0, 0)),
        ]
        args += [Watt, batt.reshape(1, 1)]
        out_specs = [out_specs, pl.BlockSpec((BM, 1), lambda i: (i, 0))]
        out_shape = [out_shape,
                     jax.ShapeDtypeStruct((N_PAD, 1), jnp.float32)]

    return pl.pallas_call(
        body,
        grid=(npb,),
        in_specs=in_specs,
        out_specs=out_specs,
        out_shape=out_shape,
    )(*args)


def _softmax(lg2d):
    def body(l_ref, o_ref):
        l = l_ref[...]
        m = jnp.max(l)
        e = jnp.exp(l - m)
        o_ref[...] = e / jnp.sum(e)

    return pl.pallas_call(
        body,
        out_shape=jax.ShapeDtypeStruct(lg2d.shape, jnp.float32),
    )(lg2d)


def kernel(x, edge_index, edge_attr, query_embedding,
           W_in, b_in, W_q, b_q,
           Wl0, bl0, Wr0, g0, be0,
           Wl1, bl1, Wr1, g1, be1,
           W_att, b_att):
    E = edge_index.shape[1]
    ch = -(-E // (NW * K))
    ch = -(-ch // 16) * 16
    tot_ch = ch * NW
    e_pad = tot_ch * K

    src = edge_index[0]
    dst = edge_index[1]
    pad = e_pad - E
    pad_ids = lax.iota(jnp.int32, pad)
    src_p = jnp.concatenate(
        [src, pad_ids % N]).reshape(tot_ch, K)
    dst_p = jnp.concatenate(
        [dst, N + pad_ids % (N_PAD - N)]).reshape(tot_ch, K)
    x_pad = jnp.pad(x, ((0, N_PAD - N), (0, 0)))
    zrow = jnp.zeros((RPT, DH), jnp.float32)

    agg = _sc_agg_kernel(tot_ch)

    h0 = _proj(x_pad, query_embedding, W_in, b_in, W_q, b_q)
    sums0 = agg(h0, src_p, dst_p, zrow)
    cnt2d = _hist(dst_p.reshape(e_pad, 1))
    cnt_col = cnt2d.reshape(-1)[:N_PAD].reshape(N_PAD, 1)
    h1 = _layer(h0, sums0, cnt_col, Wl0, bl0, Wr0, g0, be0, relu=True)
    sums1 = agg(h1, src_p, dst_p, zrow)
    h2, lg = _layer(h1, sums1, cnt_col, Wl1, bl1, Wr1, g1, be1, relu=False,
                    Watt=W_att, batt=b_att)
    w = _softmax(lg.reshape(N_PAD // DH, DH))
    return h2[:N], w.reshape(-1)[:N]


# speedup vs baseline: 7.0693x; 1.0015x over previous
import functools

import jax
import jax.numpy as jnp
from jax import lax
from jax.experimental import pallas as pl
from jax.experimental.pallas import tpu as pltpu
from jax.experimental.pallas import tpu_sc as plsc

N = 10000
DIN = 384
DH = 128
NC = 2
NS = 16
NW = NC * NS
K = 128
N_PAD = 10240
RPT = N_PAD // NS
BM = 1024


IB = 8


@functools.lru_cache(maxsize=None)
def _sc_agg_kernel(tot_ch):
    ch = tot_ch // NW
    mesh = plsc.VectorSubcoreMesh(core_axis_name="c", subcore_axis_name="s")

    nb = ch // IB

    @functools.partial(
        pl.kernel,
        out_type=jax.ShapeDtypeStruct((NC * N_PAD, DH), jnp.float32),
        mesh=mesh,
        scratch_types=[
            pltpu.VMEM_SHARED((N_PAD, DH), jnp.float32),
            pltpu.VMEM((IB, K), jnp.int32),
            pltpu.VMEM((IB, K), jnp.int32),
            pltpu.VMEM((IB, K), jnp.int32),
            pltpu.VMEM((IB, K), jnp.int32),
            pltpu.VMEM((K, DH), jnp.float32),
            pltpu.VMEM((K, DH), jnp.float32),
            pltpu.SemaphoreType.DMA,
            pltpu.SemaphoreType.DMA,
            pltpu.SemaphoreType.DMA,
            pltpu.SemaphoreType.DMA,
        ],
    )
    def agg(h_hbm, src_hbm, dst_hbm, zrow_hbm,
            sums_hbm,
            acc, srcA, dstA, srcB, dstB, rows0, rows1,
            sem0, sem1, semA, semB):
        c = lax.axis_index("c")
        s = lax.axis_index("s")
        wid = s * NC + c
        row0 = s * RPT

        base = wid * ch
        rows = (rows0, rows1)
        sems = (sem0, sem1)

        zero = pltpu.async_copy(zrow_hbm, acc.at[pl.ds(row0, RPT)], semA)
        pltpu.sync_copy(src_hbm.at[pl.ds(base, IB)], srcA)
        pltpu.sync_copy(dst_hbm.at[pl.ds(base, IB)], dstA)
        pltpu.async_copy(h_hbm.at[srcA.at[0]], rows0, sem0)
        zero.wait()
        plsc.subcore_barrier()

        def do_block(ob, Xs, Xd, Ys, Yd, semY):
            @pl.when(ob + 1 < nb)
            def _():
                off = base + (ob + 1) * IB
                pltpu.async_copy(src_hbm.at[pl.ds(off, IB)], Ys, semY)
                pltpu.async_copy(dst_hbm.at[pl.ds(off, IB)], Yd, semY)

            for t in range(IB):
                buf, sem = rows[t % 2], sems[t % 2]
                nbuf, nsem = rows[(t + 1) % 2], sems[(t + 1) % 2]
                pltpu.make_async_copy(h_hbm.at[Xs.at[t]], buf, sem).wait()
                if t + 1 < IB:
                    pltpu.async_copy(h_hbm.at[Xs.at[t + 1]], nbuf, nsem)
                else:
                    @pl.when(ob + 1 < nb)
                    def _():
                        off = base + (ob + 1) * IB
                        pltpu.make_async_copy(
                            src_hbm.at[pl.ds(off, IB)], Ys, semY).wait()
                        pltpu.make_async_copy(
                            dst_hbm.at[pl.ds(off, IB)], Yd, semY).wait()
                        pltpu.async_copy(h_hbm.at[Ys.at[0]], nbuf, nsem)
                pltpu.sync_copy(buf, acc.at[Xd.at[t]], add=True)

        def pair(p, carry):
            do_block(2 * p, srcA, dstA, srcB, dstB, semB)
            do_block(2 * p + 1, srcB, dstB, srcA, dstA, semA)
            return carry

        lax.fori_loop(0, nb // 2, pair, 0)
        plsc.subcore_barrier()

        out0 = c * N_PAD + row0
        pltpu.sync_copy(acc.at[pl.ds(row0, RPT)], sums_hbm.at[pl.ds(out0, RPT)])

    return agg


def _hist(dst_col):
    EB = 8192
    e_pad = dst_col.shape[0]

    def body(d_ref, o_ref):
        d = d_ref[...]
        hi = d // DH
        lo = d - hi * DH
        lane = lax.broadcasted_iota(jnp.int32, (1, DH), 1)
        oh_hi = (hi == lane).astype(jnp.bfloat16)
        oh_lo = (lo == lane).astype(jnp.bfloat16)
        part = lax.dot_general(oh_hi, oh_lo, (((0,), (0,)), ((), ())),
                               preferred_element_type=jnp.float32)

        @pl.when(pl.program_id(0) == 0)
        def _():
            o_ref[...] = jnp.zeros_like(o_ref)

        o_ref[...] += part

    return pl.pallas_call(
        body,
        grid=(e_pad // EB,),
        in_specs=[pl.BlockSpec((EB, 1), lambda i: (i, 0))],
        out_specs=pl.BlockSpec((DH, DH), lambda i: (0, 0)),
        out_shape=jax.ShapeDtypeStruct((DH, DH), jnp.float32),
    )(dst_col)


def _proj(x_pad, query, W_in, b_in, W_q, b_q):
    def body(x_ref, q_ref, wi_ref, bi_ref, wq_ref, bq_ref, o_ref):
        qrow = jnp.dot(q_ref[...], wq_ref[...],
                       preferred_element_type=jnp.float32) + bq_ref[...]
        o_ref[...] = (jnp.dot(x_ref[...], wi_ref[...],
                              preferred_element_type=jnp.float32)
                      + bi_ref[...] + qrow)

    return pl.pallas_call(
        body,
        grid=(N_PAD // BM,),
        in_specs=[
            pl.BlockSpec((BM, DIN), lambda i: (i, 0)),
            pl.BlockSpec((1, DIN), lambda i: (0, 0)),
            pl.BlockSpec((DIN, DH), lambda i: (0, 0)),
            pl.BlockSpec((1, DH), lambda i: (0, 0)),
            pl.BlockSpec((DIN, DH), lambda i: (0, 0)),
            pl.BlockSpec((1, DH), lambda i: (0, 0)),
        ],
        out_specs=pl.BlockSpec((BM, DH), lambda i: (i, 0)),
        out_shape=jax.ShapeDtypeStruct((N_PAD, DH), jnp.float32),
    )(x_pad, query.reshape(1, DIN), W_in, b_in.reshape(1, DH),
      W_q, b_q.reshape(1, DH))


def _layer(h, sums, cnts, Wl, bl, Wr, g, be, relu, Watt=None, batt=None):
    att = Watt is not None
    npb = N_PAD // BM

    def body(h_ref, s0_ref, s1_ref, c_ref,
             wl_ref, bl_ref, wr_ref, g_ref, be_ref, *rest):
        if att:
            watt_ref, batt_ref, o_ref, lg_ref = rest
        else:
            (o_ref,) = rest
        cnt = jnp.maximum(c_ref[...], 1.0)
        mean = (s0_ref[...] + s1_ref[...]) / cnt
        t = (jnp.dot(mean, wl_ref[...], preferred_element_type=jnp.float32)
             + bl_ref[...]
             + jnp.dot(h_ref[...], wr_ref[...],
                       preferred_element_type=jnp.float32))
        m = jnp.mean(t, axis=-1, keepdims=True)
        v = jnp.mean((t - m) * (t - m), axis=-1, keepdims=True)
        ln = (t - m) * lax.rsqrt(v + 1e-5) * g_ref[...] + be_ref[...]
        hn = h_ref[...] + ln
        if relu:
            hn = jnp.maximum(hn, 0.0)
        o_ref[...] = hn
        if att:
            lg = (jnp.dot(hn, watt_ref[...],
                          preferred_element_type=jnp.float32) + batt_ref[...])
            row = pl.program_id(0) * BM + lax.broadcasted_iota(
                jnp.int32, (BM, 1), 0)
            lg_ref[...] = jnp.where(row < N, lg, -1e30)

    in_specs = [
        pl.BlockSpec((BM, DH), lambda i: (i, 0)),
        pl.BlockSpec((BM, DH), lambda i: (i, 0)),
        pl.BlockSpec((BM, DH), lambda i: (i + npb, 0)),
        pl.BlockSpec((BM, 1), lambda i: (i, 0)),
        pl.BlockSpec((DH, DH), lambda i: (0, 0)),
        pl.BlockSpec((1, DH), lambda i: (0, 0)),
        pl.BlockSpec((DH, DH), lambda i: (0, 0)),
        pl.BlockSpec((1, DH), lambda i: (0, 0)),
        pl.BlockSpec((1, DH), lambda i: (0, 0)),
    ]
    args = [h, sums, sums, cnts, Wl, bl.reshape(1, DH), Wr,
            g.reshape(1, DH), be.reshape(1, DH)]
    out_specs = pl.BlockSpec((BM, DH), lambda i: (i, 0))
    out_shape = jax.ShapeDtypeStruct((N_PAD, DH), jnp.float32)
    if att:
        in_specs += [
            pl.BlockSpec((DH, 1), lambda i: (0, 0)),
            pl.BlockSpec((1, 1), lambda i: (0, 0)),
        ]
        args += [Watt, batt.reshape(1, 1)]
        out_specs = [out_specs, pl.BlockSpec((BM, 1), lambda i: (i, 0))]
        out_shape = [out_shape,
                     jax.ShapeDtypeStruct((N_PAD, 1), jnp.float32)]

    return pl.pallas_call(
        body,
        grid=(npb,),
        in_specs=in_specs,
        out_specs=out_specs,
        out_shape=out_shape,
    )(*args)


def _softmax(lg2d):
    def body(l_ref, o_ref):
        l = l_ref[...]
        m = jnp.max(l)
        e = jnp.exp(l - m)
        o_ref[...] = e / jnp.sum(e)

    return pl.pallas_call(
        body,
        out_shape=jax.ShapeDtypeStruct(lg2d.shape, jnp.float32),
    )(lg2d)


def kernel(x, edge_index, edge_attr, query_embedding,
           W_in, b_in, W_q, b_q,
           Wl0, bl0, Wr0, g0, be0,
           Wl1, bl1, Wr1, g1, be1,
           W_att, b_att):
    E = edge_index.shape[1]
    ch = -(-E // (NW * K))
    ch = -(-ch // 16) * 16
    tot_ch = ch * NW
    e_pad = tot_ch * K

    src = edge_index[0]
    dst = edge_index[1]
    pad = e_pad - E
    pad_ids = lax.iota(jnp.int32, pad)
    src_p = jnp.concatenate(
        [src, pad_ids % N]).reshape(tot_ch, K)
    dst_p = jnp.concatenate(
        [dst, N + pad_ids % (N_PAD - N)]).reshape(tot_ch, K)
    x_pad = jnp.pad(x, ((0, N_PAD - N), (0, 0)))
    zrow = jnp.zeros((RPT, DH), jnp.float32)

    agg = _sc_agg_kernel(tot_ch)

    h0 = _proj(x_pad, query_embedding, W_in, b_in, W_q, b_q)
    sums0 = agg(h0, src_p, dst_p, zrow)
    cnt2d = _hist(dst_p.reshape(e_pad, 1))
    cnt_col = cnt2d.reshape(-1)[:N_PAD].reshape(N_PAD, 1)
    h1 = _layer(h0, sums0, cnt_col, Wl0, bl0, Wr0, g0, be0, relu=True)
    sums1 = agg(h1, src_p, dst_p, zrow)
    h2, lg = _layer(h1, sums1, cnt_col, Wl1, bl1, Wr1, g1, be1, relu=False,
                    Watt=W_att, batt=b_att)
    w = _softmax(lg.reshape(N_PAD // DH, DH))
    return h2[:N], w.reshape(-1)[:N]
